# baseline (speedup 1.0000x reference)
import functools

import jax
import jax.numpy as jnp
from jax import lax
from jax.experimental import pallas as pl
from jax.experimental.pallas import tpu as pltpu
from jax.experimental.pallas import tpu_sc as plsc

_SCALE = 10.0
_D = 128
_NB = 4096
_S = 50
_NC = 2
_NS = 16
_NW = _NC * _NS
_BPW = _NB // _NW
_NBUF = 8
_AHEAD = 4
_LANES = 16


def _scale_buf(buf):
    def row_body(r, carry):
        for k in range(_D // _LANES):
            sl = pl.ds(k * _LANES, _LANES)
            buf[r, sl] = buf[r, sl] * _SCALE
        return carry

    lax.fori_loop(0, _S, row_body, 0, unroll=5)


_mesh = plsc.VectorSubcoreMesh(core_axis_name="c", subcore_axis_name="s")


@functools.partial(
    pl.kernel,
    out_type=jax.ShapeDtypeStruct((_NB, _S, _D), jnp.float32),
    mesh=_mesh,
    scratch_types=(
        [pltpu.VMEM((_BPW, _S), jnp.int32)]
        + [pltpu.VMEM((_S, _D), jnp.float32)] * _NBUF
        + [pltpu.SemaphoreType.DMA] * (2 * _NBUF)
    ),
)
def _gather_scale(table_hbm, idx_hbm, out_hbm, idx_v, *bufs_and_sems):
    bufs = bufs_and_sems[:_NBUF]
    gsem = bufs_and_sems[_NBUF:2 * _NBUF]
    osem = bufs_and_sems[2 * _NBUF:]
    wid = lax.axis_index("s") * _NC + lax.axis_index("c")
    pltpu.sync_copy(idx_hbm.at[pl.ds(wid * _BPW, _BPW)], idx_v)

    out_base = wid * _BPW

    def win(b):
        return idx_v.at[b, pl.ds(0, _S)]

    def gather(b, j):
        pltpu.make_async_copy(table_hbm.at[win(b)], bufs[j], gsem[j]).start()

    def scatter(b, j):
        return pltpu.make_async_copy(bufs[j], out_hbm.at[out_base + b],
                                     osem[j])

    for j in range(_AHEAD):
        gather(j, j)

    def handle(b, j):
        j4 = (j + _AHEAD) % _NBUF

        @pl.when(b >= _AHEAD)
        def _():
            scatter(b - _AHEAD, j4).wait()

        @pl.when(b + _AHEAD < _BPW)
        def _():
            gather(b + _AHEAD, j4)

        pltpu.make_async_copy(table_hbm.at[win(b)], bufs[j], gsem[j]).wait()
        _scale_buf(bufs[j])
        scatter(b, j).start()

    def body(i, carry):
        for j in range(_NBUF):
            handle(_NBUF * i + j, j)
        return carry

    lax.fori_loop(0, _BPW // _NBUF, body, 0)

    for k in range(_AHEAD):
        b = _BPW - _AHEAD + k
        scatter(b, b % _NBUF).wait()


def kernel(x, weight):
    return _gather_scale(weight, x.astype(jnp.int32))
